# baseline (speedup 1.0000x reference)
import dataclasses
import functools

import jax
import jax.numpy as jnp
from jax import lax
from jax.experimental import pallas as pl
from jax.experimental.pallas import tpu as pltpu
from jax.experimental.pallas import tpu_sc as plsc

N_USERS = 5000
N_ITEMS = 5000
N_NODES = N_USERS + N_ITEMS
EMBED = 128
HALF = EMBED // 2
N_LAYERS = 3

N_CORES = 2
N_SUBCORES = 16
LANES = 16
CHUNK = 128
GROUP = 16
N_PAD = 10240
ROWS_PER_TILE = N_PAD // N_SUBCORES
ROW_CHUNK = 32
N_ROW_CHUNKS = ROWS_PER_TILE // ROW_CHUNK
ROW_BLK = 128
N_ROW_BLKS = ROWS_PER_TILE // ROW_BLK


def _lightgcn_sc(emb2, srcs, dsts, ws):
    n_chunks = srcs.shape[1]

    mesh = plsc.VectorSubcoreMesh(
        core_axis_name="core", subcore_axis_name="subcore")

    cp = pltpu.CompilerParams()
    for fld, val in (("needs_layout_passes", False),
                     ("use_tc_tiling_on_sc", False)):
        if fld in pltpu.CompilerParams.__dataclass_fields__:
            cp = dataclasses.replace(cp, **{fld: val})

    @functools.partial(
        pl.kernel,
        out_type=jax.ShapeDtypeStruct((N_CORES, N_PAD, HALF), jnp.float32),
        mesh=mesh,
        compiler_params=cp,
        scratch_types=[
            pltpu.VMEM_SHARED((N_PAD, HALF), jnp.float32),
            pltpu.VMEM_SHARED((N_PAD, HALF), jnp.float32),
            pltpu.VMEM((GROUP, CHUNK), jnp.int32),
            pltpu.VMEM((GROUP, CHUNK), jnp.int32),
            pltpu.VMEM((GROUP * CHUNK,), jnp.float32),
            pltpu.VMEM((CHUNK, HALF), jnp.float32),
            pltpu.VMEM((CHUNK, HALF), jnp.float32),
            pltpu.VMEM((CHUNK, HALF), jnp.float32),
            pltpu.VMEM((CHUNK, HALF), jnp.float32),
            pltpu.VMEM((ROW_CHUNK, HALF), jnp.float32),
            pltpu.SemaphoreType.DMA,
            pltpu.SemaphoreType.DMA,
            pltpu.SemaphoreType.DMA,
            pltpu.SemaphoreType.DMA,
            pltpu.SemaphoreType.DMA,
            pltpu.SemaphoreType.DMA,
            pltpu.SemaphoreType.DMA,
            pltpu.SemaphoreType.DMA,
        ],
    )
    def k(emb_hbm, src_hbm, dst_hbm, w_hbm, out_hbm,
          tab_a, tab_b, src_v, dst_v, w_v, m0, m1, m2, m3, tz,
          g0, g1, g2, g3, s0, s1, s2, s3):
        c = lax.axis_index("core")
        s = lax.axis_index("subcore")
        r0 = s * ROWS_PER_TILE

        zero16 = jnp.zeros((LANES,), jnp.float32)

        @pl.loop(0, ROW_CHUNK)
        def _(r):
            for v in range(HALF // LANES):
                tz[r, pl.ds(v * LANES, LANES)] = zero16

        def fire_zeros(tab, kk, sem):
            for z in range(ROW_BLK // ROW_CHUNK):
                rows = pl.ds(r0 + kk * ROW_BLK + z * ROW_CHUNK, ROW_CHUNK)
                pltpu.async_copy(tz, tab.at[rows], sem)

        def drain_zeros(tab, kk, sem):
            for z in range(ROW_BLK // ROW_CHUNK):
                rows = pl.ds(r0 + kk * ROW_BLK + z * ROW_CHUNK, ROW_CHUNK)
                pltpu.make_async_copy(tz, tab.at[rows], sem).wait()

        def init_phase():
            bufs = (m0, m1)

            def rows_of(kk):
                return pl.ds(r0 + kk * ROW_BLK, ROW_BLK)

            def start_read(kk):
                pltpu.async_copy(
                    emb_hbm.at[c, rows_of(kk)], bufs[kk % 2], (g0, g1)[kk % 2])

            start_read(0)
            for kk in range(N_ROW_BLKS):
                b = bufs[kk % 2]
                if kk + 1 < N_ROW_BLKS:
                    if kk >= 1:
                        pltpu.make_async_copy(
                            bufs[(kk - 1) % 2], tab_a.at[rows_of(kk - 1)],
                            (g2, g3)[(kk - 1) % 2]).wait()
                        pltpu.make_async_copy(
                            bufs[(kk - 1) % 2], out_hbm.at[c, rows_of(kk - 1)],
                            (s0, s1)[(kk - 1) % 2]).wait()
                    start_read(kk + 1)
                pltpu.make_async_copy(
                    emb_hbm.at[c, rows_of(kk)], b, (g0, g1)[kk % 2]).wait()
                pltpu.async_copy(b, tab_a.at[rows_of(kk)], (g2, g3)[kk % 2])
                pltpu.async_copy(b, out_hbm.at[c, rows_of(kk)],
                                 (s0, s1)[kk % 2])
                fire_zeros(tab_b, kk, s2)
            for kk in (N_ROW_BLKS - 2, N_ROW_BLKS - 1):
                pltpu.make_async_copy(
                    bufs[kk % 2], tab_a.at[rows_of(kk)], (g2, g3)[kk % 2]).wait()
                pltpu.make_async_copy(
                    bufs[kk % 2], out_hbm.at[c, rows_of(kk)],
                    (s0, s1)[kk % 2]).wait()
            for kk in range(N_ROW_BLKS):
                drain_zeros(tab_b, kk, s2)

        init_phase()
        plsc.subcore_barrier()

        def edge_pass(cur, nxt):
            dnums = lax.GatherDimensionNumbers(
                offset_dims=(), collapsed_slice_dims=(0,),
                start_index_map=(0,))

            def scale(buf, j):
                @pl.loop(0, CHUNK // LANES)
                def _(g):
                    wrow = w_v[pl.ds(j * CHUNK + g * LANES, LANES)]
                    for kq in range(LANES):
                        wv = lax.gather(
                            wrow, jnp.full((LANES, 1), kq, jnp.int32),
                            dimension_numbers=dnums, slice_sizes=(1,),
                            mode=lax.GatherScatterMode.PROMISE_IN_BOUNDS)
                        e = g * LANES + kq
                        for v in range(HALF // LANES):
                            sl = pl.ds(v * LANES, LANES)
                            buf[e, sl] = buf[e, sl] * wv

            def start_gather(buf, sem, j):
                pltpu.async_copy(cur.at[src_v.at[j]], buf, sem)

            def wait_gather(buf, sem, j):
                pltpu.make_async_copy(cur.at[src_v.at[j]], buf, sem).wait()

            def start_scatter(buf, sem, j):
                pltpu.async_copy(buf, nxt.at[dst_v.at[j]], sem, add=True)

            def wait_scatter(buf, sem, j):
                pltpu.make_async_copy(
                    buf, nxt.at[dst_v.at[j]], sem).wait()

            @pl.loop(0, n_chunks // GROUP)
            def _(g):
                pltpu.sync_copy(src_hbm.at[s, pl.ds(g * GROUP, GROUP)], src_v)
                pltpu.sync_copy(dst_hbm.at[s, pl.ds(g * GROUP, GROUP)], dst_v)
                pltpu.sync_copy(
                    w_hbm.at[s, pl.ds(g * GROUP * CHUNK, GROUP * CHUNK)], w_v)

                bufs = ((m0, g0, s0), (m1, g1, s1), (m2, g2, s2), (m3, g3, s3))
                for q, (mb, gq, _sq) in enumerate(bufs):
                    start_gather(mb, gq, q)

                @pl.loop(0, GROUP // 4)
                def _(p):
                    j = 4 * p
                    for q, (mb, gq, sq) in enumerate(bufs):
                        wait_gather(mb, gq, j + q)
                        scale(mb, j + q)
                        start_scatter(mb, sq, j + q)

                    @pl.when(p < GROUP // 4 - 1)
                    def _():
                        for q, (mb, gq, sq) in enumerate(bufs):
                            wait_scatter(mb, sq, j + q)
                            start_gather(mb, gq, j + q + 4)

                for q, (mb, _gq, sq) in enumerate(bufs):
                    wait_scatter(mb, sq, GROUP - 4 + q)

        def inter_layer(nxt, zero=None, scale=None):
            abufs = (m0, m1)
            obufs = (m2, m3)

            def rows_of(kk):
                return pl.ds(r0 + kk * ROW_BLK, ROW_BLK)

            def start_reads(kk):
                pltpu.async_copy(
                    nxt.at[rows_of(kk)], abufs[kk % 2], (g0, g1)[kk % 2])
                pltpu.async_copy(
                    out_hbm.at[c, rows_of(kk)], obufs[kk % 2], (s0, s1)[kk % 2])

            start_reads(0)
            for kk in range(N_ROW_BLKS):
                ab = abufs[kk % 2]
                ob = obufs[kk % 2]
                if kk + 1 < N_ROW_BLKS:
                    if kk >= 1:
                        pltpu.make_async_copy(
                            obufs[(kk - 1) % 2],
                            out_hbm.at[c, rows_of(kk - 1)],
                            (g2, g3)[(kk - 1) % 2]).wait()
                    start_reads(kk + 1)
                pltpu.make_async_copy(
                    nxt.at[rows_of(kk)], ab, (g0, g1)[kk % 2]).wait()
                pltpu.make_async_copy(
                    out_hbm.at[c, rows_of(kk)], ob, (s0, s1)[kk % 2]).wait()

                @pl.loop(0, ROW_BLK, unroll=4)
                def _(r):
                    for v in range(HALF // LANES):
                        sl = pl.ds(v * LANES, LANES)
                        val = ob[r, sl] + ab[r, sl]
                        if scale is not None:
                            val = val * scale
                        ob[r, sl] = val

                pltpu.async_copy(ob, out_hbm.at[c, rows_of(kk)],
                                 (g2, g3)[kk % 2])
                if zero is not None:
                    fire_zeros(zero, kk, s2)
            for kk in (N_ROW_BLKS - 2, N_ROW_BLKS - 1):
                pltpu.make_async_copy(
                    obufs[kk % 2], out_hbm.at[c, rows_of(kk)],
                    (g2, g3)[kk % 2]).wait()
            if zero is not None:
                for kk in range(N_ROW_BLKS):
                    drain_zeros(zero, kk, s2)

        edge_pass(tab_a, tab_b)
        plsc.subcore_barrier()
        inter_layer(tab_b, zero=tab_a)
        plsc.subcore_barrier()

        edge_pass(tab_b, tab_a)
        plsc.subcore_barrier()
        inter_layer(tab_a, zero=tab_b)
        plsc.subcore_barrier()

        edge_pass(tab_a, tab_b)
        plsc.subcore_barrier()
        inter_layer(tab_b, scale=0.25)

    return k(emb2, srcs, dsts, ws)


def kernel(edge_index, edge_values, user_emb, item_emb):
    n_edges = edge_values.shape[0]
    step = GROUP * CHUNK
    per_tile = -(-n_edges // (N_SUBCORES * step)) * step
    n_pad = N_SUBCORES * per_tile - n_edges

    dst = edge_index[0].astype(jnp.int32)
    src = edge_index[1].astype(jnp.int32)
    w = edge_values.astype(jnp.float32)
    if n_pad:
        zpad = jnp.zeros((n_pad,), jnp.int32)
        dst = jnp.concatenate([dst, zpad])
        src = jnp.concatenate([src, zpad])
        w = jnp.concatenate([w, jnp.zeros((n_pad,), jnp.float32)])

    srcs = src.reshape(N_SUBCORES, per_tile // CHUNK, CHUNK)
    dsts = dst.reshape(N_SUBCORES, per_tile // CHUNK, CHUNK)
    ws = w.reshape(N_SUBCORES, per_tile)

    all_emb = jnp.concatenate([
        user_emb, item_emb,
        jnp.zeros((N_PAD - N_NODES, EMBED), jnp.float32)], axis=0)
    emb2 = all_emb.reshape(N_PAD, N_CORES, HALF).transpose(1, 0, 2)

    out = _lightgcn_sc(emb2, srcs, dsts, ws)
    res = out.transpose(1, 0, 2).reshape(N_PAD, EMBED)
    return (res[:N_USERS], res[N_USERS:N_NODES])


# speedup vs baseline: 1.8056x; 1.8056x over previous
import dataclasses
import functools

import jax
import jax.numpy as jnp
from jax import lax
from jax.experimental import pallas as pl
from jax.experimental.pallas import tpu as pltpu
from jax.experimental.pallas import tpu_sc as plsc

N_USERS = 5000
N_ITEMS = 5000
N_NODES = N_USERS + N_ITEMS
EMBED = 128
HALF = EMBED // 2
N_LAYERS = 3

N_CORES = 2
N_SUBCORES = 16
LANES = 16
CHUNK = 128
GROUP = 32
N_PAD = 10240
ROWS_PER_TILE = N_PAD // N_SUBCORES
ROW_CHUNK = 32
N_ROW_CHUNKS = ROWS_PER_TILE // ROW_CHUNK
ROW_BLK = 128
N_ROW_BLKS = ROWS_PER_TILE // ROW_BLK


def _lightgcn_sc(emb2, srcs, dsts, ws):
    n_chunks = srcs.shape[1]

    mesh = plsc.VectorSubcoreMesh(
        core_axis_name="core", subcore_axis_name="subcore")

    cp = pltpu.CompilerParams()
    for fld, val in (("needs_layout_passes", False),
                     ("use_tc_tiling_on_sc", False)):
        if fld in pltpu.CompilerParams.__dataclass_fields__:
            cp = dataclasses.replace(cp, **{fld: val})

    @functools.partial(
        pl.kernel,
        out_type=jax.ShapeDtypeStruct((N_CORES, N_PAD, HALF), jnp.float32),
        mesh=mesh,
        compiler_params=cp,
        scratch_types=[
            pltpu.VMEM_SHARED((N_PAD, HALF), jnp.float32),
            pltpu.VMEM_SHARED((N_PAD, HALF), jnp.float32),
            pltpu.VMEM((GROUP, CHUNK), jnp.int32),
            pltpu.VMEM((GROUP, CHUNK), jnp.int32),
            pltpu.VMEM((GROUP * CHUNK,), jnp.float32),
            pltpu.VMEM((CHUNK, HALF), jnp.float32),
            pltpu.VMEM((CHUNK, HALF), jnp.float32),
            pltpu.VMEM((CHUNK, HALF), jnp.float32),
            pltpu.VMEM((CHUNK, HALF), jnp.float32),
            pltpu.VMEM((ROW_CHUNK, HALF), jnp.float32),
            pltpu.SemaphoreType.DMA,
            pltpu.SemaphoreType.DMA,
            pltpu.SemaphoreType.DMA,
            pltpu.SemaphoreType.DMA,
            pltpu.SemaphoreType.DMA,
            pltpu.SemaphoreType.DMA,
            pltpu.SemaphoreType.DMA,
            pltpu.SemaphoreType.DMA,
        ],
    )
    def k(emb_hbm, src_hbm, dst_hbm, w_hbm, out_hbm,
          tab_a, tab_b, src_v, dst_v, w_v, m0, m1, m2, m3, tz,
          g0, g1, g2, g3, s0, s1, s2, s3):
        c = lax.axis_index("core")
        s = lax.axis_index("subcore")
        r0 = s * ROWS_PER_TILE

        zero16 = jnp.zeros((LANES,), jnp.float32)

        @pl.loop(0, ROW_CHUNK)
        def _(r):
            for v in range(HALF // LANES):
                tz[r, pl.ds(v * LANES, LANES)] = zero16

        def fire_zeros(tab, kk, sem):
            for z in range(ROW_BLK // ROW_CHUNK):
                rows = pl.ds(r0 + kk * ROW_BLK + z * ROW_CHUNK, ROW_CHUNK)
                pltpu.async_copy(tz, tab.at[rows], sem)

        def drain_zeros(tab, kk, sem):
            for z in range(ROW_BLK // ROW_CHUNK):
                rows = pl.ds(r0 + kk * ROW_BLK + z * ROW_CHUNK, ROW_CHUNK)
                pltpu.make_async_copy(tz, tab.at[rows], sem).wait()

        def init_phase():
            bufs = (m0, m1)

            def rows_of(kk):
                return pl.ds(r0 + kk * ROW_BLK, ROW_BLK)

            def start_read(kk):
                pltpu.async_copy(
                    emb_hbm.at[c, rows_of(kk)], bufs[kk % 2], (g0, g1)[kk % 2])

            start_read(0)
            for kk in range(N_ROW_BLKS):
                b = bufs[kk % 2]
                if kk + 1 < N_ROW_BLKS:
                    if kk >= 1:
                        pltpu.make_async_copy(
                            bufs[(kk - 1) % 2], tab_a.at[rows_of(kk - 1)],
                            (g2, g3)[(kk - 1) % 2]).wait()
                        pltpu.make_async_copy(
                            bufs[(kk - 1) % 2], out_hbm.at[c, rows_of(kk - 1)],
                            (s0, s1)[(kk - 1) % 2]).wait()
                    start_read(kk + 1)
                pltpu.make_async_copy(
                    emb_hbm.at[c, rows_of(kk)], b, (g0, g1)[kk % 2]).wait()
                pltpu.async_copy(b, tab_a.at[rows_of(kk)], (g2, g3)[kk % 2])
                pltpu.async_copy(b, out_hbm.at[c, rows_of(kk)],
                                 (s0, s1)[kk % 2])
                fire_zeros(tab_b, kk, s2)
            for kk in (N_ROW_BLKS - 2, N_ROW_BLKS - 1):
                pltpu.make_async_copy(
                    bufs[kk % 2], tab_a.at[rows_of(kk)], (g2, g3)[kk % 2]).wait()
                pltpu.make_async_copy(
                    bufs[kk % 2], out_hbm.at[c, rows_of(kk)],
                    (s0, s1)[kk % 2]).wait()
            for kk in range(N_ROW_BLKS):
                drain_zeros(tab_b, kk, s2)

        init_phase()
        plsc.subcore_barrier()

        def edge_pass(cur, nxt):
            def scale(buf, j):
                @pl.loop(0, CHUNK, unroll=8)
                def _(e):
                    wv = plsc.load_gather(
                        w_v, [jnp.full((LANES,), j * CHUNK + e, jnp.int32)])
                    for v in range(HALF // LANES):
                        sl = pl.ds(v * LANES, LANES)
                        buf[e, sl] = buf[e, sl] * wv

            def start_gather(buf, sem, j):
                pltpu.async_copy(cur.at[src_v.at[j]], buf, sem)

            def wait_gather(buf, sem, j):
                pltpu.make_async_copy(cur.at[src_v.at[j]], buf, sem).wait()

            def start_scatter(buf, sem, j):
                pltpu.async_copy(buf, nxt.at[dst_v.at[j]], sem, add=True)

            def wait_scatter(buf, sem, j):
                pltpu.make_async_copy(
                    buf, nxt.at[dst_v.at[j]], sem).wait()

            @pl.loop(0, n_chunks // GROUP)
            def _(g):
                pltpu.sync_copy(src_hbm.at[s, pl.ds(g * GROUP, GROUP)], src_v)
                pltpu.sync_copy(dst_hbm.at[s, pl.ds(g * GROUP, GROUP)], dst_v)
                pltpu.sync_copy(
                    w_hbm.at[s, pl.ds(g * GROUP * CHUNK, GROUP * CHUNK)], w_v)

                bufs = ((m0, g0, s0), (m1, g1, s1), (m2, g2, s2), (m3, g3, s3))
                for q, (mb, gq, _sq) in enumerate(bufs):
                    start_gather(mb, gq, q)

                @pl.loop(0, GROUP // 4)
                def _(p):
                    j = 4 * p
                    for q, (mb, gq, sq) in enumerate(bufs):
                        wait_gather(mb, gq, j + q)
                        scale(mb, j + q)
                        start_scatter(mb, sq, j + q)

                    @pl.when(p < GROUP // 4 - 1)
                    def _():
                        for q, (mb, gq, sq) in enumerate(bufs):
                            wait_scatter(mb, sq, j + q)
                            start_gather(mb, gq, j + q + 4)

                for q, (mb, _gq, sq) in enumerate(bufs):
                    wait_scatter(mb, sq, GROUP - 4 + q)

        def inter_layer(nxt, zero=None, scale=None):
            abufs = (m0, m1)
            obufs = (m2, m3)

            def rows_of(kk):
                return pl.ds(r0 + kk * ROW_BLK, ROW_BLK)

            def start_reads(kk):
                pltpu.async_copy(
                    nxt.at[rows_of(kk)], abufs[kk % 2], (g0, g1)[kk % 2])
                pltpu.async_copy(
                    out_hbm.at[c, rows_of(kk)], obufs[kk % 2], (s0, s1)[kk % 2])

            start_reads(0)
            for kk in range(N_ROW_BLKS):
                ab = abufs[kk % 2]
                ob = obufs[kk % 2]
                if kk + 1 < N_ROW_BLKS:
                    if kk >= 1:
                        pltpu.make_async_copy(
                            obufs[(kk - 1) % 2],
                            out_hbm.at[c, rows_of(kk - 1)],
                            (g2, g3)[(kk - 1) % 2]).wait()
                    start_reads(kk + 1)
                pltpu.make_async_copy(
                    nxt.at[rows_of(kk)], ab, (g0, g1)[kk % 2]).wait()
                pltpu.make_async_copy(
                    out_hbm.at[c, rows_of(kk)], ob, (s0, s1)[kk % 2]).wait()

                @pl.loop(0, ROW_BLK, unroll=4)
                def _(r):
                    for v in range(HALF // LANES):
                        sl = pl.ds(v * LANES, LANES)
                        val = ob[r, sl] + ab[r, sl]
                        if scale is not None:
                            val = val * scale
                        ob[r, sl] = val

                pltpu.async_copy(ob, out_hbm.at[c, rows_of(kk)],
                                 (g2, g3)[kk % 2])
                if zero is not None:
                    fire_zeros(zero, kk, s2)
            for kk in (N_ROW_BLKS - 2, N_ROW_BLKS - 1):
                pltpu.make_async_copy(
                    obufs[kk % 2], out_hbm.at[c, rows_of(kk)],
                    (g2, g3)[kk % 2]).wait()
            if zero is not None:
                for kk in range(N_ROW_BLKS):
                    drain_zeros(zero, kk, s2)

        edge_pass(tab_a, tab_b)
        plsc.subcore_barrier()
        inter_layer(tab_b, zero=tab_a)
        plsc.subcore_barrier()

        edge_pass(tab_b, tab_a)
        plsc.subcore_barrier()
        inter_layer(tab_a, zero=tab_b)
        plsc.subcore_barrier()

        edge_pass(tab_a, tab_b)
        plsc.subcore_barrier()
        inter_layer(tab_b, scale=0.25)

    return k(emb2, srcs, dsts, ws)


def kernel(edge_index, edge_values, user_emb, item_emb):
    n_edges = edge_values.shape[0]
    step = GROUP * CHUNK
    per_tile = -(-n_edges // (N_SUBCORES * step)) * step
    n_pad = N_SUBCORES * per_tile - n_edges

    dst = edge_index[0].astype(jnp.int32)
    src = edge_index[1].astype(jnp.int32)
    w = edge_values.astype(jnp.float32)
    if n_pad:
        zpad = jnp.zeros((n_pad,), jnp.int32)
        dst = jnp.concatenate([dst, zpad])
        src = jnp.concatenate([src, zpad])
        w = jnp.concatenate([w, jnp.zeros((n_pad,), jnp.float32)])

    srcs = src.reshape(N_SUBCORES, per_tile // CHUNK, CHUNK)
    dsts = dst.reshape(N_SUBCORES, per_tile // CHUNK, CHUNK)
    ws = w.reshape(N_SUBCORES, per_tile)

    all_emb = jnp.concatenate([
        user_emb, item_emb,
        jnp.zeros((N_PAD - N_NODES, EMBED), jnp.float32)], axis=0)
    emb2 = all_emb.reshape(N_PAD, N_CORES, HALF).transpose(1, 0, 2)

    out = _lightgcn_sc(emb2, srcs, dsts, ws)
    res = out.transpose(1, 0, 2).reshape(N_PAD, EMBED)
    return (res[:N_USERS], res[N_USERS:N_NODES])
